# baseline (speedup 1.0000x reference)
import functools

import jax
import jax.numpy as jnp
from jax import lax
from jax.experimental import pallas as pl
from jax.experimental.pallas import tpu as pltpu
from jax.experimental.pallas import tpu_sc as plsc

N_NODES = 10000
E_EDGES = 320000
D = 128
R = 8
NLAYERS = 6
EDIM = 16
B = 64
K = 32

NC = 2
NS = 16
NW = NC * NS
CH = 128
TPB = 10240
NCHUNKS = TPB // CH
EP = NW * TPB
NPAD = N_NODES + 112
RPT = NPAD // NS
AW = 128
BK = B * K
BKT = BK // NW

BN = 400
GRID_N = N_NODES // BN

@functools.cache
def _mesh():
    return plsc.VectorSubcoreMesh(
        core_axis_name="c", subcore_axis_name="s", num_cores=NC, num_subcores=NS)

_RPT_CHUNKS = [(0, 128), (128, 128), (256, 128), (384, 128), (512, 120)]


def _zero_buf(buf, width):
    z = jnp.zeros((16,), jnp.float32)

    @pl.loop(0, CH)
    def _(i):
        for j in range(width // 16):
            buf[i, pl.ds(j * 16, 16)] = z


def _acc_zero_and_sync(acc, buf, width, sid):
    _zero_buf(buf, width)
    row0 = sid * RPT
    for off, sz in _RPT_CHUNKS:
        pltpu.sync_copy(buf.at[pl.ds(0, sz)], acc.at[pl.ds(row0 + off, sz)])
    plsc.subcore_barrier()


GRP = 8
SUB = 2
ROWS_PT = TPB // CH
NOUT = ROWS_PT // GRP


@functools.cache
def _sc_seg():
    @functools.partial(
        pl.kernel,
        out_type=jax.ShapeDtypeStruct((NC, NPAD, D), jnp.float32),
        mesh=_mesh(),
        scratch_types=[
            pltpu.VMEM((GRP, CH), jnp.int32),
            pltpu.VMEM((GRP, CH), jnp.int32),
            pltpu.VMEM((CH, D), jnp.float32),
            pltpu.VMEM((CH, D), jnp.float32),
            pltpu.VMEM_SHARED((NPAD, D), jnp.float32),
            pltpu.SemaphoreType.DMA,
            pltpu.SemaphoreType.DMA,
        ],
    )
    def k(table_hbm, eidx_hbm, dst_hbm, out_hbm, gid2, dst2, r0, r1,
          acc, s0, s1):
        rows = [r0, r1]
        sems = [s0, s1]
        cid = lax.axis_index("c")
        sid = lax.axis_index("s")
        wid = sid * NC + cid
        _acc_zero_and_sync(acc, r0, D, sid)
        rowbase = wid * ROWS_PT

        @pl.loop(0, NOUT)
        def _(t):
            row = rowbase + t * GRP
            pltpu.sync_copy(eidx_hbm.at[pl.ds(row, GRP)], gid2)
            pltpu.sync_copy(dst_hbm.at[pl.ds(row, GRP)], dst2)
            cps = [None] * GRP
            for c in range(SUB):
                cps[c] = pltpu.async_copy(
                    table_hbm.at[gid2.at[c]], rows[c % SUB], sems[c % SUB])
            for c in range(GRP):
                cps[c].wait()
                pltpu.sync_copy(rows[c % SUB], acc.at[dst2.at[c]], add=True)
                if c + SUB < GRP:
                    cps[c + SUB] = pltpu.async_copy(
                        table_hbm.at[gid2.at[c + SUB]], rows[c % SUB],
                        sems[c % SUB])

        plsc.subcore_barrier()
        row0 = sid * RPT
        prev = [None, None]
        for i, (off, sz) in enumerate(_RPT_CHUNKS):
            b = rows[i % 2]
            if prev[i % 2] is not None:
                prev[i % 2].wait()
            pltpu.sync_copy(acc.at[pl.ds(row0 + off, sz)], b.at[pl.ds(0, sz)])
            prev[i % 2] = pltpu.async_copy(
                b.at[pl.ds(0, sz)], out_hbm.at[cid, pl.ds(row0 + off, sz)],
                sems[i % 2])
        for p in prev:
            if p is not None:
                p.wait()

    return k


@functools.cache
def _sc_aug():
    @functools.partial(
        pl.kernel,
        out_type=jax.ShapeDtypeStruct((NC, NPAD, D), jnp.float32),
        mesh=_mesh(),
        scratch_types=[
            pltpu.VMEM((GRP, CH), jnp.int32),
            pltpu.VMEM((CH // 4, D), jnp.float32),
            pltpu.VMEM((CH // 4, D), jnp.float32),
            pltpu.VMEM((CH, D), jnp.float32),
            pltpu.VMEM((CH, D), jnp.float32),
            pltpu.VMEM_SHARED((NPAD, D), jnp.float32),
            pltpu.SemaphoreType.DMA,
            pltpu.SemaphoreType.DMA,
            pltpu.SemaphoreType.DMA,
            pltpu.SemaphoreType.DMA,
        ],
    )
    def k(aug4_hbm, dst_hbm, out_hbm, dst2, pk0, pk1, st0, st1, acc,
          sp0, sp1, ss0, ss1):
        pks = [pk0, pk1]
        sts = [st0, st1]
        psem = [sp0, sp1]
        ssem = [ss0, ss1]
        cid = lax.axis_index("c")
        sid = lax.axis_index("s")
        wid = sid * NC + cid
        _acc_zero_and_sync(acc, st0, D, sid)
        _zero_buf(st1, D)
        rowbase = wid * ROWS_PT

        @pl.loop(0, NOUT)
        def _(t):
            row = rowbase + t * GRP
            pltpu.sync_copy(dst_hbm.at[pl.ds(row, GRP)], dst2)
            cp = [None] * GRP
            cs = [None] * GRP
            for c in range(2):
                cp[c] = pltpu.async_copy(
                    aug4_hbm.at[pl.ds((row + c) * (CH // 4), CH // 4)],
                    pks[c % 2], psem[c % 2])
            for c in range(GRP):
                b = c % 2
                cp[c].wait()
                if c >= 2:
                    cs[c - 2].wait()
                pk = pks[b]
                st = sts[b]

                @pl.loop(0, CH // 4)
                def _(r):
                    for g in range(4):
                        st[r * 4 + g, pl.ds(0, 16)] = pk[r, pl.ds(g * 32, 16)]
                        st[r * 4 + g, pl.ds(16, 16)] = pk[r, pl.ds(g * 32 + 16, 16)]
                cs[c] = pltpu.async_copy(st, acc.at[dst2.at[c]], ssem[b],
                                         add=True)
                if c + 2 < GRP:
                    cp[c + 2] = pltpu.async_copy(
                        aug4_hbm.at[pl.ds((row + c + 2) * (CH // 4), CH // 4)],
                        pks[b], psem[b])
            cs[GRP - 2].wait()
            cs[GRP - 1].wait()

        plsc.subcore_barrier()
        row0 = sid * RPT
        for off, sz in _RPT_CHUNKS:
            pltpu.sync_copy(acc.at[pl.ds(row0 + off, sz)], st0.at[pl.ds(0, sz)])
            pltpu.sync_copy(st0.at[pl.ds(0, sz)],
                            out_hbm.at[cid, pl.ds(row0 + off, sz)])

    return k


@functools.cache
def _sc_head():
    @functools.partial(
        pl.kernel,
        out_type=(
            jax.ShapeDtypeStruct((BK, D), jnp.float32),
            jax.ShapeDtypeStruct((BK, D), jnp.float32),
        ),
        mesh=_mesh(),
        scratch_types=[
            pltpu.VMEM((BKT,), jnp.int32),
            pltpu.VMEM((BKT, D), jnp.float32),
            pltpu.SemaphoreType.DMA,
        ],
    )
    def k(x_hbm, si_hbm, ti_hbm, so_hbm, to_hbm, idx_v, rows_v, sem):
        cid = lax.axis_index("c")
        sid = lax.axis_index("s")
        wid = sid * NC + cid
        base = wid * BKT
        pltpu.sync_copy(si_hbm.at[pl.ds(base, BKT)], idx_v)
        pltpu.async_copy(x_hbm.at[idx_v], rows_v, sem).wait()
        pltpu.sync_copy(rows_v, so_hbm.at[pl.ds(base, BKT)])
        pltpu.sync_copy(ti_hbm.at[pl.ds(base, BKT)], idx_v)
        pltpu.async_copy(x_hbm.at[idx_v], rows_v, sem).wait()
        pltpu.sync_copy(rows_v, to_hbm.at[pl.ds(base, BKT)])

    return k


def _layernorm(out, lns, lnb):
    mu = jnp.mean(out, axis=1, keepdims=True)
    var = jnp.mean((out - mu) ** 2, axis=1, keepdims=True)
    return (out - mu) * lax.rsqrt(var + 1e-5) * lns + lnb


def _tc0_body(naug_ref, wrel0_ref, wself0_ref, wedge0_ref, bias_ref, lns_ref,
              lnb_ref, wreln_ref, x_ref, trans_ref):
    p = naug_ref[0] + naug_ref[1]
    eagg = p[:, :EDIM]
    cntr = p[:, EDIM:EDIM + R]
    inv = 1.0 / jnp.maximum(jnp.sum(cntr, axis=1, keepdims=True), 1.0)
    m0 = jnp.sum(wrel0_ref[...], axis=1)
    agg = jnp.dot(cntr, m0, preferred_element_type=jnp.float32)
    agg = agg + jnp.dot(eagg, wedge0_ref[...], preferred_element_type=jnp.float32)
    out = agg * inv + jnp.sum(wself0_ref[...], axis=0, keepdims=True) + bias_ref[...]
    x = jnp.maximum(_layernorm(out, lns_ref[...], lnb_ref[...]), 0.0) + 1.0
    x_ref[...] = x
    for r in range(R):
        trans_ref[r] = jnp.dot(x, wreln_ref[r], preferred_element_type=jnp.float32)


def _tcl_body(has_next, part_ref, naug_ref, x_ref, wself_ref, wedge_ref,
              bias_ref, lns_ref, lnb_ref, *rest):
    if has_next:
        wreln_ref, xo_ref, trans_ref = rest
    else:
        wreln_ref, trans_ref = None, None
        (xo_ref,) = rest
    p = naug_ref[0] + naug_ref[1]
    eagg = p[:, :EDIM]
    cntr = p[:, EDIM:EDIM + R]
    inv = 1.0 / jnp.maximum(jnp.sum(cntr, axis=1, keepdims=True), 1.0)
    s = part_ref[0] + part_ref[1]
    agg = s + jnp.dot(eagg, wedge_ref[...], preferred_element_type=jnp.float32)
    x = x_ref[...]
    out = agg * inv + jnp.dot(x, wself_ref[...], preferred_element_type=jnp.float32) + bias_ref[...]
    xn = jnp.maximum(_layernorm(out, lns_ref[...], lnb_ref[...]), 0.0) + x
    xo_ref[...] = xn
    if has_next:
        for r in range(R):
            trans_ref[r] = jnp.dot(xn, wreln_ref[r], preferred_element_type=jnp.float32)


def _full3(shape):
    return pl.BlockSpec(shape, lambda i: tuple(0 for _ in shape))


def _tc0(naug, wrel0, wself0, wedge0, bias0, lns0, lnb0, wrel1):
    return pl.pallas_call(
        _tc0_body,
        grid=(GRID_N,),
        in_specs=[
            pl.BlockSpec((NC, BN, AW), lambda i: (0, i, 0)),
            _full3((R, D, D)),
            _full3((D, D)),
            _full3((EDIM, D)),
            _full3((1, D)),
            _full3((1, D)),
            _full3((1, D)),
            _full3((R, D, D)),
        ],
        out_specs=[
            pl.BlockSpec((BN, D), lambda i: (i, 0)),
            pl.BlockSpec((R, BN, D), lambda i: (0, i, 0)),
        ],
        out_shape=[
            jax.ShapeDtypeStruct((N_NODES, D), jnp.float32),
            jax.ShapeDtypeStruct((R, N_NODES, D), jnp.float32),
        ],
    )(naug, wrel0, wself0, wedge0, bias0, lns0, lnb0, wrel1)


def _tc_layer(part, naug, x, wself, wedge, biasl, lns, lnb, wreln=None):
    has_next = wreln is not None
    in_specs = [
        pl.BlockSpec((NC, BN, D), lambda i: (0, i, 0)),
        pl.BlockSpec((NC, BN, AW), lambda i: (0, i, 0)),
        pl.BlockSpec((BN, D), lambda i: (i, 0)),
        _full3((D, D)),
        _full3((EDIM, D)),
        _full3((1, D)),
        _full3((1, D)),
        _full3((1, D)),
    ]
    out_specs = [pl.BlockSpec((BN, D), lambda i: (i, 0))]
    out_shape = [jax.ShapeDtypeStruct((N_NODES, D), jnp.float32)]
    args = [part, naug, x, wself, wedge, biasl, lns, lnb]
    if has_next:
        in_specs.append(_full3((R, D, D)))
        out_specs.append(pl.BlockSpec((R, BN, D), lambda i: (0, i, 0)))
        out_shape.append(jax.ShapeDtypeStruct((R, N_NODES, D), jnp.float32))
        args.append(wreln)
    return pl.pallas_call(
        functools.partial(_tcl_body, has_next),
        grid=(GRID_N,),
        in_specs=in_specs,
        out_specs=out_specs,
        out_shape=out_shape,
    )(*args)


QP = EP // 4
PBLK = 2560
_NEBLK = E_EDGES // PBLK - 1


def _pack_body(e0, e1, e2, e3, t0, t1, t2, t3, out_ref):
    i = pl.program_id(0)
    iota8 = lax.iota(jnp.int32, R)[None, :]
    pieces = []
    for a, (eref, tref) in enumerate(((e0, t0), (e1, t1), (e2, t2), (e3, t3))):
        base = i * PBLK + a * QP
        gr = base + lax.broadcasted_iota(jnp.int32, (PBLK, 1), 0)
        valid = (gr < E_EDGES).astype(jnp.float32)
        pieces.append(eref[...] * valid)
        pieces.append((tref[...] == iota8).astype(jnp.float32) * valid)
        pieces.append(jnp.zeros((PBLK, 8), jnp.float32))
    out_ref[...] = jnp.concatenate(pieces, axis=1)


def _tc_pack(emb, et1):
    def _emap(a):
        return lambda i: (jnp.minimum(i + a * (QP // PBLK), _NEBLK), 0)

    espec = [pl.BlockSpec((PBLK, EDIM), _emap(a)) for a in range(4)]
    tspec = [pl.BlockSpec((PBLK, 1), _emap(a)) for a in range(4)]
    return pl.pallas_call(
        _pack_body,
        grid=(QP // PBLK,),
        in_specs=espec + tspec,
        out_specs=pl.BlockSpec((PBLK, D), lambda i: (i, 0)),
        out_shape=jax.ShapeDtypeStruct((QP, D), jnp.float32),
    )(emb, emb, emb, emb, et1, et1, et1, et1)


def _tc_head_body(se_ref, te_ref, oh_ref, rel_ref, out_ref):
    rel_e = jnp.dot(oh_ref[...], rel_ref[...], preferred_element_type=jnp.float32)
    out_ref[...] = jnp.sum(se_ref[...] * te_ref[...] * rel_e, axis=1, keepdims=True)


def _tc_head(se, te, oh, rel_emb):
    return pl.pallas_call(
        _tc_head_body,
        grid=(1,),
        in_specs=[
            _full3((BK, D)),
            _full3((BK, D)),
            _full3((BK, R)),
            _full3((R, D)),
        ],
        out_specs=pl.BlockSpec((BK, 1), lambda i: (0, 0)),
        out_shape=jax.ShapeDtypeStruct((BK, 1), jnp.float32),
    )(se, te, oh, rel_emb)


def kernel(edge_embeddings, Wrel, Wself, bias, Wedge, ln_scale, ln_beta,
           rel_emb, edge_index, edge_type, batch):
    f32 = jnp.float32
    src = edge_index[0]
    dst = edge_index[1]
    et = edge_type
    eidx = et * N_NODES + src

    npad_extra = EP - E_EDGES
    ar = lax.iota(jnp.int32, npad_extra)
    eidx_p = jnp.concatenate([eidx, (ar * 97) % (R * N_NODES)])
    dst_p = jnp.concatenate([dst, N_NODES + (ar % (NPAD - N_NODES))])

    eidx2 = eidx_p.reshape(EP // CH, CH)
    dst2 = dst_p.reshape(EP // CH, CH)
    aug4 = _tc_pack(edge_embeddings, et[:, None])
    dstp4 = dst_p.reshape(4, QP).transpose(1, 0).reshape(EP // CH, CH)
    naug = _sc_aug()(aug4, dstp4)

    x, trans = _tc0(naug, Wrel[0], Wself[0], Wedge[0], bias[0][None],
                    ln_scale[0][None], ln_beta[0][None], Wrel[1])
    for l in range(1, NLAYERS):
        part = _sc_seg()(trans.reshape(R * N_NODES, D), eidx2, dst2)
        if l < NLAYERS - 1:
            x, trans = _tc_layer(part, naug, x, Wself[l], Wedge[l], bias[l][None],
                                 ln_scale[l][None], ln_beta[l][None], Wrel[l + 1])
        else:
            (x,) = _tc_layer(part, naug, x, Wself[l], Wedge[l], bias[l][None],
                             ln_scale[l][None], ln_beta[l][None])

    bs = batch[:, :, 0].reshape(-1)
    bt = batch[:, :, 1].reshape(-1)
    br = batch[:, :, 2].reshape(-1)
    se, te = _sc_head()(x, bs, bt)
    oh = (br[:, None] == lax.iota(jnp.int32, R)[None, :]).astype(f32)
    score = _tc_head(se, te, oh, rel_emb)
    return score[:, 0].reshape(B, K)


# speedup vs baseline: 1.1200x; 1.1200x over previous
import functools

import jax
import jax.numpy as jnp
from jax import lax
from jax.experimental import pallas as pl
from jax.experimental.pallas import tpu as pltpu
from jax.experimental.pallas import tpu_sc as plsc

N_NODES = 10000
E_EDGES = 320000
D = 128
R = 8
NLAYERS = 6
EDIM = 16
B = 64
K = 32

NC = 2
NS = 16
NW = NC * NS
CH = 128
TPB = 10240
NCHUNKS = TPB // CH
EP = NW * TPB
NPAD = N_NODES + 112
RPT = NPAD // NS
AW = 128
BK = B * K
BKT = BK // NW

BN = 400
GRID_N = N_NODES // BN

@functools.cache
def _mesh():
    return plsc.VectorSubcoreMesh(
        core_axis_name="c", subcore_axis_name="s", num_cores=NC, num_subcores=NS)

_RPT_CHUNKS = [(0, 128), (128, 128), (256, 128), (384, 128), (512, 120)]


def _zero_buf(buf, width):
    z = jnp.zeros((16,), jnp.float32)

    @pl.loop(0, CH)
    def _(i):
        for j in range(width // 16):
            buf[i, pl.ds(j * 16, 16)] = z


def _acc_zero_and_sync(acc, buf, width, sid):
    _zero_buf(buf, width)
    row0 = sid * RPT
    for off, sz in _RPT_CHUNKS:
        pltpu.sync_copy(buf.at[pl.ds(0, sz)], acc.at[pl.ds(row0 + off, sz)])
    plsc.subcore_barrier()


GRP = 8
SUB = 2
ROWS_PT = TPB // CH
NOUT = ROWS_PT // GRP


@functools.cache
def _sc_seg():
    @functools.partial(
        pl.kernel,
        out_type=jax.ShapeDtypeStruct((NC, NPAD, D), jnp.float32),
        mesh=_mesh(),
        scratch_types=[
            pltpu.VMEM((GRP, CH), jnp.int32),
            pltpu.VMEM((GRP, CH), jnp.int32),
            pltpu.VMEM((CH, D), jnp.float32),
            pltpu.VMEM((CH, D), jnp.float32),
            pltpu.VMEM_SHARED((NPAD, D), jnp.float32),
            pltpu.SemaphoreType.DMA,
            pltpu.SemaphoreType.DMA,
        ],
    )
    def k(table_hbm, eidx_hbm, dst_hbm, out_hbm, gid2, dst2, r0, r1,
          acc, s0, s1):
        rows = [r0, r1]
        sems = [s0, s1]
        cid = lax.axis_index("c")
        sid = lax.axis_index("s")
        wid = sid * NC + cid
        _acc_zero_and_sync(acc, r0, D, sid)
        rowbase = wid * ROWS_PT

        @pl.loop(0, NOUT)
        def _(t):
            row = rowbase + t * GRP
            pltpu.sync_copy(eidx_hbm.at[pl.ds(row, GRP)], gid2)
            pltpu.sync_copy(dst_hbm.at[pl.ds(row, GRP)], dst2)
            cps = [None] * GRP
            for c in range(SUB):
                cps[c] = pltpu.async_copy(
                    table_hbm.at[gid2.at[c]], rows[c % SUB], sems[c % SUB])
            for c in range(GRP):
                cps[c].wait()
                pltpu.sync_copy(rows[c % SUB], acc.at[dst2.at[c]], add=True)
                if c + SUB < GRP:
                    cps[c + SUB] = pltpu.async_copy(
                        table_hbm.at[gid2.at[c + SUB]], rows[c % SUB],
                        sems[c % SUB])

        plsc.subcore_barrier()
        row0 = sid * RPT
        prev = [None, None]
        for i, (off, sz) in enumerate(_RPT_CHUNKS):
            b = rows[i % 2]
            if prev[i % 2] is not None:
                prev[i % 2].wait()
            pltpu.sync_copy(acc.at[pl.ds(row0 + off, sz)], b.at[pl.ds(0, sz)])
            prev[i % 2] = pltpu.async_copy(
                b.at[pl.ds(0, sz)], out_hbm.at[cid, pl.ds(row0 + off, sz)],
                sems[i % 2])
        for p in prev:
            if p is not None:
                p.wait()

    return k


@functools.cache
def _sc_aug():
    @functools.partial(
        pl.kernel,
        out_type=jax.ShapeDtypeStruct((NC, NPAD, D), jnp.float32),
        mesh=_mesh(),
        scratch_types=[
            pltpu.VMEM((GRP, CH), jnp.int32),
            pltpu.VMEM((CH // 4, D), jnp.float32),
            pltpu.VMEM((CH // 4, D), jnp.float32),
            pltpu.VMEM((CH, D), jnp.float32),
            pltpu.VMEM((CH, D), jnp.float32),
            pltpu.VMEM_SHARED((NPAD, D), jnp.float32),
            pltpu.SemaphoreType.DMA,
            pltpu.SemaphoreType.DMA,
            pltpu.SemaphoreType.DMA,
            pltpu.SemaphoreType.DMA,
        ],
    )
    def k(aug4_hbm, dst_hbm, out_hbm, dst2, pk0, pk1, st0, st1, acc,
          sp0, sp1, ss0, ss1):
        pks = [pk0, pk1]
        sts = [st0, st1]
        psem = [sp0, sp1]
        ssem = [ss0, ss1]
        cid = lax.axis_index("c")
        sid = lax.axis_index("s")
        wid = sid * NC + cid
        _acc_zero_and_sync(acc, st0, D, sid)
        _zero_buf(st1, D)
        rowbase = wid * ROWS_PT

        @pl.loop(0, NOUT)
        def _(t):
            row = rowbase + t * GRP
            pltpu.sync_copy(dst_hbm.at[pl.ds(row, GRP)], dst2)
            cp = [None] * GRP
            cs = [None] * GRP
            for c in range(2):
                cp[c] = pltpu.async_copy(
                    aug4_hbm.at[pl.ds((row + c) * (CH // 4), CH // 4)],
                    pks[c % 2], psem[c % 2])
            for c in range(GRP):
                b = c % 2
                cp[c].wait()
                if c >= 2:
                    cs[c - 2].wait()
                pk = pks[b]
                st = sts[b]

                @pl.loop(0, CH // 4)
                def _(r):
                    for g in range(4):
                        st[r * 4 + g, pl.ds(0, 16)] = pk[r, pl.ds(g * 32, 16)]
                        st[r * 4 + g, pl.ds(16, 16)] = pk[r, pl.ds(g * 32 + 16, 16)]
                cs[c] = pltpu.async_copy(st, acc.at[dst2.at[c]], ssem[b],
                                         add=True)
                if c + 2 < GRP:
                    cp[c + 2] = pltpu.async_copy(
                        aug4_hbm.at[pl.ds((row + c + 2) * (CH // 4), CH // 4)],
                        pks[b], psem[b])
            cs[GRP - 2].wait()
            cs[GRP - 1].wait()

        plsc.subcore_barrier()
        row0 = sid * RPT
        for off, sz in _RPT_CHUNKS:
            pltpu.sync_copy(acc.at[pl.ds(row0 + off, sz)], st0.at[pl.ds(0, sz)])
            pltpu.sync_copy(st0.at[pl.ds(0, sz)],
                            out_hbm.at[cid, pl.ds(row0 + off, sz)])

    return k


@functools.cache
def _sc_head():
    @functools.partial(
        pl.kernel,
        out_type=(
            jax.ShapeDtypeStruct((BK, D), jnp.float32),
            jax.ShapeDtypeStruct((BK, D), jnp.float32),
        ),
        mesh=_mesh(),
        scratch_types=[
            pltpu.VMEM((BKT,), jnp.int32),
            pltpu.VMEM((BKT, D), jnp.float32),
            pltpu.SemaphoreType.DMA,
        ],
    )
    def k(x_hbm, si_hbm, ti_hbm, so_hbm, to_hbm, idx_v, rows_v, sem):
        cid = lax.axis_index("c")
        sid = lax.axis_index("s")
        wid = sid * NC + cid
        base = wid * BKT
        pltpu.sync_copy(si_hbm.at[pl.ds(base, BKT)], idx_v)
        pltpu.async_copy(x_hbm.at[idx_v], rows_v, sem).wait()
        pltpu.sync_copy(rows_v, so_hbm.at[pl.ds(base, BKT)])
        pltpu.sync_copy(ti_hbm.at[pl.ds(base, BKT)], idx_v)
        pltpu.async_copy(x_hbm.at[idx_v], rows_v, sem).wait()
        pltpu.sync_copy(rows_v, to_hbm.at[pl.ds(base, BKT)])

    return k


def _layernorm(out, lns, lnb):
    mu = jnp.mean(out, axis=1, keepdims=True)
    var = jnp.mean((out - mu) ** 2, axis=1, keepdims=True)
    return (out - mu) * lax.rsqrt(var + 1e-5) * lns + lnb


def _tc0_body(part_ref, naug_ref, wself0_ref, wedge0_ref, bias_ref, lns_ref,
              lnb_ref, wreln_ref, x_ref, trans_ref):
    p = naug_ref[0] + naug_ref[1]
    eagg = p[:, :EDIM]
    inv = 1.0 / jnp.maximum(p[:, EDIM:EDIM + 1], 1.0)
    s = part_ref[0] + part_ref[1]
    agg = s + jnp.dot(eagg, wedge0_ref[...], preferred_element_type=jnp.float32)
    out = agg * inv + jnp.sum(wself0_ref[...], axis=0, keepdims=True) + bias_ref[...]
    x = jnp.maximum(_layernorm(out, lns_ref[...], lnb_ref[...]), 0.0) + 1.0
    x_ref[...] = x
    for r in range(R):
        trans_ref[r] = jnp.dot(x, wreln_ref[r], preferred_element_type=jnp.float32)


def _tcl_body(has_next, part_ref, naug_ref, x_ref, wself_ref, wedge_ref,
              bias_ref, lns_ref, lnb_ref, *rest):
    if has_next:
        wreln_ref, xo_ref, trans_ref = rest
    else:
        wreln_ref, trans_ref = None, None
        (xo_ref,) = rest
    p = naug_ref[0] + naug_ref[1]
    eagg = p[:, :EDIM]
    inv = 1.0 / jnp.maximum(p[:, EDIM:EDIM + 1], 1.0)
    s = part_ref[0] + part_ref[1]
    agg = s + jnp.dot(eagg, wedge_ref[...], preferred_element_type=jnp.float32)
    x = x_ref[...]
    out = agg * inv + jnp.dot(x, wself_ref[...], preferred_element_type=jnp.float32) + bias_ref[...]
    xn = jnp.maximum(_layernorm(out, lns_ref[...], lnb_ref[...]), 0.0) + x
    xo_ref[...] = xn
    if has_next:
        for r in range(R):
            trans_ref[r] = jnp.dot(xn, wreln_ref[r], preferred_element_type=jnp.float32)


def _full3(shape):
    return pl.BlockSpec(shape, lambda i: tuple(0 for _ in shape))


def _tc0(part, naug, wself0, wedge0, bias0, lns0, lnb0, wrel1):
    return pl.pallas_call(
        _tc0_body,
        grid=(GRID_N,),
        in_specs=[
            pl.BlockSpec((NC, BN, D), lambda i: (0, i, 0)),
            pl.BlockSpec((NC, BN, AW), lambda i: (0, i, 0)),
            _full3((D, D)),
            _full3((EDIM, D)),
            _full3((1, D)),
            _full3((1, D)),
            _full3((1, D)),
            _full3((R, D, D)),
        ],
        out_specs=[
            pl.BlockSpec((BN, D), lambda i: (i, 0)),
            pl.BlockSpec((R, BN, D), lambda i: (0, i, 0)),
        ],
        out_shape=[
            jax.ShapeDtypeStruct((N_NODES, D), jnp.float32),
            jax.ShapeDtypeStruct((R, N_NODES, D), jnp.float32),
        ],
    )(part, naug, wself0, wedge0, bias0, lns0, lnb0, wrel1)


def _tc_layer(part, naug, x, wself, wedge, biasl, lns, lnb, wreln=None):
    has_next = wreln is not None
    in_specs = [
        pl.BlockSpec((NC, BN, D), lambda i: (0, i, 0)),
        pl.BlockSpec((NC, BN, AW), lambda i: (0, i, 0)),
        pl.BlockSpec((BN, D), lambda i: (i, 0)),
        _full3((D, D)),
        _full3((EDIM, D)),
        _full3((1, D)),
        _full3((1, D)),
        _full3((1, D)),
    ]
    out_specs = [pl.BlockSpec((BN, D), lambda i: (i, 0))]
    out_shape = [jax.ShapeDtypeStruct((N_NODES, D), jnp.float32)]
    args = [part, naug, x, wself, wedge, biasl, lns, lnb]
    if has_next:
        in_specs.append(_full3((R, D, D)))
        out_specs.append(pl.BlockSpec((R, BN, D), lambda i: (0, i, 0)))
        out_shape.append(jax.ShapeDtypeStruct((R, N_NODES, D), jnp.float32))
        args.append(wreln)
    return pl.pallas_call(
        functools.partial(_tcl_body, has_next),
        grid=(GRID_N,),
        in_specs=in_specs,
        out_specs=out_specs,
        out_shape=out_shape,
    )(*args)


QP = EP // 4
PBLK = 2560
_NEBLK = E_EDGES // PBLK - 1


def _pack_body(e0, e1, e2, e3, out_ref):
    i = pl.program_id(0)
    pieces = []
    for a, eref in enumerate((e0, e1, e2, e3)):
        base = i * PBLK + a * QP
        gr = base + lax.broadcasted_iota(jnp.int32, (PBLK, 1), 0)
        valid = (gr < E_EDGES).astype(jnp.float32)
        pieces.append(eref[...] * valid)
        pieces.append(valid)
        pieces.append(jnp.zeros((PBLK, 15), jnp.float32))
    out_ref[...] = jnp.concatenate(pieces, axis=1)


def _tc_pack(emb):
    def _emap(a):
        return lambda i: (jnp.minimum(i + a * (QP // PBLK), _NEBLK), 0)

    espec = [pl.BlockSpec((PBLK, EDIM), _emap(a)) for a in range(4)]
    return pl.pallas_call(
        _pack_body,
        grid=(QP // PBLK,),
        in_specs=espec,
        out_specs=pl.BlockSpec((PBLK, D), lambda i: (i, 0)),
        out_shape=jax.ShapeDtypeStruct((QP, D), jnp.float32),
    )(emb, emb, emb, emb)


MREP = 1024


def _m0_body(w_ref, out_ref):
    s = jnp.sum(w_ref[0], axis=0, keepdims=True)
    out_ref[...] = jnp.broadcast_to(s, (MREP, D))


def _tc_m0(wrel0):
    return pl.pallas_call(
        _m0_body,
        grid=(R,),
        in_specs=[pl.BlockSpec((1, D, D), lambda r: (r, 0, 0))],
        out_specs=pl.BlockSpec((MREP, D), lambda r: (r, 0)),
        out_shape=jax.ShapeDtypeStruct((R * MREP, D), jnp.float32),
    )(wrel0)


def _tc_head_body(se_ref, te_ref, oh_ref, rel_ref, out_ref):
    rel_e = jnp.dot(oh_ref[...], rel_ref[...], preferred_element_type=jnp.float32)
    out_ref[...] = jnp.sum(se_ref[...] * te_ref[...] * rel_e, axis=1, keepdims=True)


def _tc_head(se, te, oh, rel_emb):
    return pl.pallas_call(
        _tc_head_body,
        grid=(1,),
        in_specs=[
            _full3((BK, D)),
            _full3((BK, D)),
            _full3((BK, R)),
            _full3((R, D)),
        ],
        out_specs=pl.BlockSpec((BK, 1), lambda i: (0, 0)),
        out_shape=jax.ShapeDtypeStruct((BK, 1), jnp.float32),
    )(se, te, oh, rel_emb)


def kernel(edge_embeddings, Wrel, Wself, bias, Wedge, ln_scale, ln_beta,
           rel_emb, edge_index, edge_type, batch):
    f32 = jnp.float32
    src = edge_index[0]
    dst = edge_index[1]
    et = edge_type
    eidx = et * N_NODES + src

    npad_extra = EP - E_EDGES
    ar = lax.iota(jnp.int32, npad_extra)
    eidx_p = jnp.concatenate([eidx, (ar * 97) % (R * N_NODES)])
    dst_p = jnp.concatenate([dst, N_NODES + (ar % (NPAD - N_NODES))])

    eidx2 = eidx_p.reshape(EP // CH, CH)
    dst2 = dst_p.reshape(EP // CH, CH)
    aug4 = _tc_pack(edge_embeddings)
    dstp4 = dst_p.reshape(4, QP).transpose(1, 0).reshape(EP // CH, CH)
    naug = _sc_aug()(aug4, dstp4)

    m0t = _tc_m0(Wrel[0])
    idx0 = et * MREP + (lax.iota(jnp.int32, E_EDGES) % MREP)
    idx02 = jnp.concatenate(
        [idx0, (ar * 97) % (R * MREP)]).reshape(EP // CH, CH)
    part0 = _sc_seg()(m0t, idx02, dst2)
    x, trans = _tc0(part0, naug, Wself[0], Wedge[0], bias[0][None],
                    ln_scale[0][None], ln_beta[0][None], Wrel[1])
    for l in range(1, NLAYERS):
        part = _sc_seg()(trans.reshape(R * N_NODES, D), eidx2, dst2)
        if l < NLAYERS - 1:
            x, trans = _tc_layer(part, naug, x, Wself[l], Wedge[l], bias[l][None],
                                 ln_scale[l][None], ln_beta[l][None], Wrel[l + 1])
        else:
            (x,) = _tc_layer(part, naug, x, Wself[l], Wedge[l], bias[l][None],
                             ln_scale[l][None], ln_beta[l][None])

    bs = batch[:, :, 0].reshape(-1)
    bt = batch[:, :, 1].reshape(-1)
    br = batch[:, :, 2].reshape(-1)
    se, te = _sc_head()(x, bs, bt)
    oh = (br[:, None] == lax.iota(jnp.int32, R)[None, :]).astype(f32)
    score = _tc_head(se, te, oh, rel_emb)
    return score[:, 0].reshape(B, K)


# speedup vs baseline: 1.2440x; 1.1107x over previous
import functools

import jax
import jax.numpy as jnp
from jax import lax
from jax.experimental import pallas as pl
from jax.experimental.pallas import tpu as pltpu
from jax.experimental.pallas import tpu_sc as plsc

N_NODES = 10000
E_EDGES = 320000
D = 128
R = 8
NLAYERS = 6
EDIM = 16
B = 64
K = 32

NC = 2
NS = 16
NW = NC * NS
CH = 128
TPB = 10240
NCHUNKS = TPB // CH
EP = NW * TPB
NPAD = N_NODES + 112
RPT = NPAD // NS
AW = 128
BK = B * K
BKT = BK // NW

BN = 400
GRID_N = N_NODES // BN

@functools.cache
def _mesh():
    return plsc.VectorSubcoreMesh(
        core_axis_name="c", subcore_axis_name="s", num_cores=NC, num_subcores=NS)

_RPT_CHUNKS = [(0, 128), (128, 128), (256, 128), (384, 128), (512, 120)]


def _zero_buf(buf, width):
    z = jnp.zeros((16,), jnp.float32)

    @pl.loop(0, CH)
    def _(i):
        for j in range(width // 16):
            buf[i, pl.ds(j * 16, 16)] = z


def _acc_zero_and_sync(acc, buf, width, sid):
    _zero_buf(buf, width)
    row0 = sid * RPT
    for off, sz in _RPT_CHUNKS:
        pltpu.sync_copy(buf.at[pl.ds(0, sz)], acc.at[pl.ds(row0 + off, sz)])
    plsc.subcore_barrier()


GRP = 16
SUB = 2
ROWS_PT = TPB // CH
NOUT = ROWS_PT // GRP


@functools.cache
def _sc_seg():
    @functools.partial(
        pl.kernel,
        out_type=jax.ShapeDtypeStruct((NC, NPAD, D), jnp.float32),
        mesh=_mesh(),
        scratch_types=[
            pltpu.VMEM((GRP, CH), jnp.int32),
            pltpu.VMEM((GRP, CH), jnp.int32),
            pltpu.VMEM((CH, D), jnp.float32),
            pltpu.VMEM((CH, D), jnp.float32),
            pltpu.VMEM_SHARED((NPAD, D), jnp.float32),
            pltpu.SemaphoreType.DMA,
            pltpu.SemaphoreType.DMA,
        ],
    )
    def k(table_hbm, eidx_hbm, dst_hbm, out_hbm, gid2, dst2, r0, r1,
          acc, s0, s1):
        rows = [r0, r1]
        sems = [s0, s1]
        cid = lax.axis_index("c")
        sid = lax.axis_index("s")
        wid = sid * NC + cid
        _acc_zero_and_sync(acc, r0, D, sid)
        rowbase = wid * ROWS_PT

        @pl.loop(0, NOUT)
        def _(t):
            row = rowbase + t * GRP
            pltpu.sync_copy(eidx_hbm.at[pl.ds(row, GRP)], gid2)
            pltpu.sync_copy(dst_hbm.at[pl.ds(row, GRP)], dst2)
            cps = [None] * GRP
            for c in range(SUB):
                cps[c] = pltpu.async_copy(
                    table_hbm.at[gid2.at[c]], rows[c % SUB], sems[c % SUB])
            for c in range(GRP):
                cps[c].wait()
                pltpu.sync_copy(rows[c % SUB], acc.at[dst2.at[c]], add=True)
                if c + SUB < GRP:
                    cps[c + SUB] = pltpu.async_copy(
                        table_hbm.at[gid2.at[c + SUB]], rows[c % SUB],
                        sems[c % SUB])

        plsc.subcore_barrier()
        row0 = sid * RPT
        prev = [None, None]
        for i, (off, sz) in enumerate(_RPT_CHUNKS):
            b = rows[i % 2]
            if prev[i % 2] is not None:
                prev[i % 2].wait()
            pltpu.sync_copy(acc.at[pl.ds(row0 + off, sz)], b.at[pl.ds(0, sz)])
            prev[i % 2] = pltpu.async_copy(
                b.at[pl.ds(0, sz)], out_hbm.at[cid, pl.ds(row0 + off, sz)],
                sems[i % 2])
        for p in prev:
            if p is not None:
                p.wait()

    return k


@functools.cache
def _sc_aug():
    @functools.partial(
        pl.kernel,
        out_type=jax.ShapeDtypeStruct((NC, NPAD, D), jnp.float32),
        mesh=_mesh(),
        scratch_types=[
            pltpu.VMEM((GRP, CH), jnp.int32),
            pltpu.VMEM((CH // 4, D), jnp.float32),
            pltpu.VMEM((CH // 4, D), jnp.float32),
            pltpu.VMEM((CH, D), jnp.float32),
            pltpu.VMEM((CH, D), jnp.float32),
            pltpu.VMEM_SHARED((NPAD, D), jnp.float32),
            pltpu.SemaphoreType.DMA,
            pltpu.SemaphoreType.DMA,
            pltpu.SemaphoreType.DMA,
            pltpu.SemaphoreType.DMA,
        ],
    )
    def k(aug4_hbm, dst_hbm, out_hbm, dst2, pk0, pk1, st0, st1, acc,
          sp0, sp1, ss0, ss1):
        pks = [pk0, pk1]
        sts = [st0, st1]
        psem = [sp0, sp1]
        ssem = [ss0, ss1]
        cid = lax.axis_index("c")
        sid = lax.axis_index("s")
        wid = sid * NC + cid
        _acc_zero_and_sync(acc, st0, D, sid)
        _zero_buf(st1, D)
        rowbase = wid * ROWS_PT

        @pl.loop(0, NOUT)
        def _(t):
            row = rowbase + t * GRP
            pltpu.sync_copy(dst_hbm.at[pl.ds(row, GRP)], dst2)
            cp = [None] * GRP
            cs = [None] * GRP
            for c in range(2):
                cp[c] = pltpu.async_copy(
                    aug4_hbm.at[pl.ds((row + c) * (CH // 4), CH // 4)],
                    pks[c % 2], psem[c % 2])
            for c in range(GRP):
                b = c % 2
                cp[c].wait()
                if c >= 2:
                    cs[c - 2].wait()
                pk = pks[b]
                st = sts[b]

                @pl.loop(0, CH // 4)
                def _(r):
                    for a in range(4):
                        st[a * 32 + r, pl.ds(0, 16)] = pk[r, pl.ds(a * 32, 16)]
                        st[a * 32 + r, pl.ds(16, 16)] = pk[r, pl.ds(a * 32 + 16, 16)]
                cs[c] = pltpu.async_copy(st, acc.at[dst2.at[c]], ssem[b],
                                         add=True)
                if c + 2 < GRP:
                    cp[c + 2] = pltpu.async_copy(
                        aug4_hbm.at[pl.ds((row + c + 2) * (CH // 4), CH // 4)],
                        pks[b], psem[b])
            cs[GRP - 2].wait()
            cs[GRP - 1].wait()

        plsc.subcore_barrier()
        row0 = sid * RPT
        for off, sz in _RPT_CHUNKS:
            pltpu.sync_copy(acc.at[pl.ds(row0 + off, sz)], st0.at[pl.ds(0, sz)])
            pltpu.sync_copy(st0.at[pl.ds(0, sz)],
                            out_hbm.at[cid, pl.ds(row0 + off, sz)])

    return k


@functools.cache
def _sc_head():
    @functools.partial(
        pl.kernel,
        out_type=(
            jax.ShapeDtypeStruct((BK, D), jnp.float32),
            jax.ShapeDtypeStruct((BK, D), jnp.float32),
        ),
        mesh=_mesh(),
        scratch_types=[
            pltpu.VMEM((BKT,), jnp.int32),
            pltpu.VMEM((BKT, D), jnp.float32),
            pltpu.SemaphoreType.DMA,
        ],
    )
    def k(x_hbm, si_hbm, ti_hbm, so_hbm, to_hbm, idx_v, rows_v, sem):
        cid = lax.axis_index("c")
        sid = lax.axis_index("s")
        wid = sid * NC + cid
        base = wid * BKT
        pltpu.sync_copy(si_hbm.at[pl.ds(base, BKT)], idx_v)
        pltpu.async_copy(x_hbm.at[idx_v], rows_v, sem).wait()
        pltpu.sync_copy(rows_v, so_hbm.at[pl.ds(base, BKT)])
        pltpu.sync_copy(ti_hbm.at[pl.ds(base, BKT)], idx_v)
        pltpu.async_copy(x_hbm.at[idx_v], rows_v, sem).wait()
        pltpu.sync_copy(rows_v, to_hbm.at[pl.ds(base, BKT)])

    return k


def _layernorm(out, lns, lnb):
    mu = jnp.mean(out, axis=1, keepdims=True)
    var = jnp.mean((out - mu) ** 2, axis=1, keepdims=True)
    return (out - mu) * lax.rsqrt(var + 1e-5) * lns + lnb


def _tc0_body(part_ref, naug_ref, wself0_ref, wedge0_ref, bias_ref, lns_ref,
              lnb_ref, wreln_ref, x_ref, trans_ref):
    p = naug_ref[0] + naug_ref[1]
    eagg = p[:, :EDIM]
    inv = 1.0 / jnp.maximum(p[:, EDIM:EDIM + 1], 1.0)
    s = part_ref[0] + part_ref[1]
    agg = s + jnp.dot(eagg, wedge0_ref[...], preferred_element_type=jnp.float32)
    out = agg * inv + jnp.sum(wself0_ref[...], axis=0, keepdims=True) + bias_ref[...]
    x = jnp.maximum(_layernorm(out, lns_ref[...], lnb_ref[...]), 0.0) + 1.0
    x_ref[...] = x
    for r in range(R):
        trans_ref[r] = jnp.dot(x, wreln_ref[r], preferred_element_type=jnp.float32)


def _tcl_body(has_next, part_ref, naug_ref, x_ref, wself_ref, wedge_ref,
              bias_ref, lns_ref, lnb_ref, *rest):
    if has_next:
        wreln_ref, xo_ref, trans_ref = rest
    else:
        wreln_ref, trans_ref = None, None
        (xo_ref,) = rest
    p = naug_ref[0] + naug_ref[1]
    eagg = p[:, :EDIM]
    inv = 1.0 / jnp.maximum(p[:, EDIM:EDIM + 1], 1.0)
    s = part_ref[0] + part_ref[1]
    agg = s + jnp.dot(eagg, wedge_ref[...], preferred_element_type=jnp.float32)
    x = x_ref[...]
    out = agg * inv + jnp.dot(x, wself_ref[...], preferred_element_type=jnp.float32) + bias_ref[...]
    xn = jnp.maximum(_layernorm(out, lns_ref[...], lnb_ref[...]), 0.0) + x
    xo_ref[...] = xn
    if has_next:
        for r in range(R):
            trans_ref[r] = jnp.dot(xn, wreln_ref[r], preferred_element_type=jnp.float32)


def _full3(shape):
    return pl.BlockSpec(shape, lambda i: tuple(0 for _ in shape))


def _tc0(part, naug, wself0, wedge0, bias0, lns0, lnb0, wrel1):
    return pl.pallas_call(
        _tc0_body,
        grid=(GRID_N,),
        in_specs=[
            pl.BlockSpec((NC, BN, D), lambda i: (0, i, 0)),
            pl.BlockSpec((NC, BN, AW), lambda i: (0, i, 0)),
            _full3((D, D)),
            _full3((EDIM, D)),
            _full3((1, D)),
            _full3((1, D)),
            _full3((1, D)),
            _full3((R, D, D)),
        ],
        out_specs=[
            pl.BlockSpec((BN, D), lambda i: (i, 0)),
            pl.BlockSpec((R, BN, D), lambda i: (0, i, 0)),
        ],
        out_shape=[
            jax.ShapeDtypeStruct((N_NODES, D), jnp.float32),
            jax.ShapeDtypeStruct((R, N_NODES, D), jnp.float32),
        ],
    )(part, naug, wself0, wedge0, bias0, lns0, lnb0, wrel1)


def _tc_layer(part, naug, x, wself, wedge, biasl, lns, lnb, wreln=None):
    has_next = wreln is not None
    in_specs = [
        pl.BlockSpec((NC, BN, D), lambda i: (0, i, 0)),
        pl.BlockSpec((NC, BN, AW), lambda i: (0, i, 0)),
        pl.BlockSpec((BN, D), lambda i: (i, 0)),
        _full3((D, D)),
        _full3((EDIM, D)),
        _full3((1, D)),
        _full3((1, D)),
        _full3((1, D)),
    ]
    out_specs = [pl.BlockSpec((BN, D), lambda i: (i, 0))]
    out_shape = [jax.ShapeDtypeStruct((N_NODES, D), jnp.float32)]
    args = [part, naug, x, wself, wedge, biasl, lns, lnb]
    if has_next:
        in_specs.append(_full3((R, D, D)))
        out_specs.append(pl.BlockSpec((R, BN, D), lambda i: (0, i, 0)))
        out_shape.append(jax.ShapeDtypeStruct((R, N_NODES, D), jnp.float32))
        args.append(wreln)
    return pl.pallas_call(
        functools.partial(_tcl_body, has_next),
        grid=(GRID_N,),
        in_specs=in_specs,
        out_specs=out_specs,
        out_shape=out_shape,
    )(*args)


QP = EP // 4
PBLK = 2560
_NEBLK = E_EDGES // PBLK - 1


def _pack_body(e0, e1, e2, e3, out_ref):
    i = pl.program_id(0)
    pieces = []
    for a, eref in enumerate((e0, e1, e2, e3)):
        base = i * PBLK + a * QP
        gr = base + lax.broadcasted_iota(jnp.int32, (PBLK, 1), 0)
        valid = (gr < E_EDGES).astype(jnp.float32)
        pieces.append(eref[...] * valid)
        pieces.append(valid)
        pieces.append(jnp.zeros((PBLK, 15), jnp.float32))
    out_ref[...] = jnp.concatenate(pieces, axis=1)


def _tc_pack(emb):
    def _emap(a):
        return lambda i: (jnp.minimum(i + a * (QP // PBLK), _NEBLK), 0)

    espec = [pl.BlockSpec((PBLK, EDIM), _emap(a)) for a in range(4)]
    return pl.pallas_call(
        _pack_body,
        grid=(QP // PBLK,),
        in_specs=espec,
        out_specs=pl.BlockSpec((PBLK, D), lambda i: (i, 0)),
        out_shape=jax.ShapeDtypeStruct((QP, D), jnp.float32),
    )(emb, emb, emb, emb)


MREP = 1024


def _m0_body(w_ref, out_ref):
    s = jnp.sum(w_ref[0], axis=0, keepdims=True)
    out_ref[...] = jnp.broadcast_to(s, (MREP, D))


def _tc_m0(wrel0):
    return pl.pallas_call(
        _m0_body,
        grid=(R,),
        in_specs=[pl.BlockSpec((1, D, D), lambda r: (r, 0, 0))],
        out_specs=pl.BlockSpec((MREP, D), lambda r: (r, 0)),
        out_shape=jax.ShapeDtypeStruct((R * MREP, D), jnp.float32),
    )(wrel0)


def _tc_head_body(se_ref, te_ref, oh_ref, rel_ref, out_ref):
    rel_e = jnp.dot(oh_ref[...], rel_ref[...], preferred_element_type=jnp.float32)
    out_ref[...] = jnp.sum(se_ref[...] * te_ref[...] * rel_e, axis=1, keepdims=True)


def _tc_head(se, te, oh, rel_emb):
    return pl.pallas_call(
        _tc_head_body,
        grid=(1,),
        in_specs=[
            _full3((BK, D)),
            _full3((BK, D)),
            _full3((BK, R)),
            _full3((R, D)),
        ],
        out_specs=pl.BlockSpec((BK, 1), lambda i: (0, 0)),
        out_shape=jax.ShapeDtypeStruct((BK, 1), jnp.float32),
    )(se, te, oh, rel_emb)


def kernel(edge_embeddings, Wrel, Wself, bias, Wedge, ln_scale, ln_beta,
           rel_emb, edge_index, edge_type, batch):
    f32 = jnp.float32
    src = edge_index[0]
    dst = edge_index[1]
    et = edge_type
    eidx = et * N_NODES + src

    npad_extra = EP - E_EDGES
    ar = lax.iota(jnp.int32, npad_extra)
    eidx_p = jnp.concatenate([eidx, (ar * 97) % (R * N_NODES)])
    dst_p = jnp.concatenate([dst, N_NODES + (ar % (NPAD - N_NODES))])

    eidx2 = eidx_p.reshape(EP // CH, CH)
    dst2 = dst_p.reshape(EP // CH, CH)
    aug4 = _tc_pack(edge_embeddings)
    dstp4 = dst_p.reshape(4, EP // CH, CH // 4).transpose(1, 0, 2).reshape(
        EP // CH, CH)
    naug = _sc_aug()(aug4, dstp4)

    m0t = _tc_m0(Wrel[0])
    idx0 = et * MREP + (lax.iota(jnp.int32, E_EDGES) % MREP)
    idx02 = jnp.concatenate(
        [idx0, (ar * 97) % (R * MREP)]).reshape(EP // CH, CH)
    part0 = _sc_seg()(m0t, idx02, dst2)
    x, trans = _tc0(part0, naug, Wself[0], Wedge[0], bias[0][None],
                    ln_scale[0][None], ln_beta[0][None], Wrel[1])
    for l in range(1, NLAYERS):
        part = _sc_seg()(trans.reshape(R * N_NODES, D), eidx2, dst2)
        if l < NLAYERS - 1:
            x, trans = _tc_layer(part, naug, x, Wself[l], Wedge[l], bias[l][None],
                                 ln_scale[l][None], ln_beta[l][None], Wrel[l + 1])
        else:
            (x,) = _tc_layer(part, naug, x, Wself[l], Wedge[l], bias[l][None],
                             ln_scale[l][None], ln_beta[l][None])

    bs = batch[:, :, 0].reshape(-1)
    bt = batch[:, :, 1].reshape(-1)
    br = batch[:, :, 2].reshape(-1)
    se, te = _sc_head()(x, bs, bt)
    oh = (br[:, None] == lax.iota(jnp.int32, R)[None, :]).astype(f32)
    score = _tc_head(se, te, oh, rel_emb)
    return score[:, 0].reshape(B, K)


# speedup vs baseline: 1.2448x; 1.0007x over previous
import functools

import jax
import jax.numpy as jnp
from jax import lax
from jax.experimental import pallas as pl
from jax.experimental.pallas import tpu as pltpu
from jax.experimental.pallas import tpu_sc as plsc

N_NODES = 10000
E_EDGES = 320000
D = 128
R = 8
NLAYERS = 6
EDIM = 16
B = 64
K = 32

NC = 2
NS = 16
NW = NC * NS
CH = 128
TPB = 10240
EP = NW * TPB
NPAD = N_NODES + 112
RPT = NPAD // NS
AW = 128
BK = B * K
BKT = BK // NW

BN = 400
GRID_N = N_NODES // BN

@functools.cache
def _mesh():
    return plsc.VectorSubcoreMesh(
        core_axis_name="c", subcore_axis_name="s", num_cores=NC, num_subcores=NS)

_RPT_CHUNKS = [(0, 128), (128, 128), (256, 128), (384, 128), (512, 120)]


def _zero_buf(buf, width):
    z = jnp.zeros((16,), jnp.float32)

    @pl.loop(0, CH)
    def _(i):
        for j in range(width // 16):
            buf[i, pl.ds(j * 16, 16)] = z


def _acc_zero_and_sync(acc, buf, width, sid):
    _zero_buf(buf, width)
    row0 = sid * RPT
    for off, sz in _RPT_CHUNKS:
        pltpu.sync_copy(buf.at[pl.ds(0, sz)], acc.at[pl.ds(row0 + off, sz)])
    plsc.subcore_barrier()


GRP = 16
SUB = 2
ROWS_PT = TPB // CH
NOUT = ROWS_PT // GRP


@functools.cache
def _sc_seg():
    @functools.partial(
        pl.kernel,
        out_type=jax.ShapeDtypeStruct((NC, NPAD, D), jnp.float32),
        mesh=_mesh(),
        scratch_types=[
            pltpu.VMEM((GRP, CH), jnp.int32),
            pltpu.VMEM((GRP, CH), jnp.int32),
            pltpu.VMEM((CH, D), jnp.float32),
            pltpu.VMEM((CH, D), jnp.float32),
            pltpu.VMEM_SHARED((NPAD, D), jnp.float32),
            pltpu.SemaphoreType.DMA,
            pltpu.SemaphoreType.DMA,
        ],
    )
    def k(table_hbm, eidx_hbm, dst_hbm, out_hbm, gid2, dst2, r0, r1,
          acc, s0, s1):
        rows = [r0, r1]
        sems = [s0, s1]
        cid = lax.axis_index("c")
        sid = lax.axis_index("s")
        wid = sid * NC + cid
        _acc_zero_and_sync(acc, r0, D, sid)
        rowbase = wid * ROWS_PT

        @pl.loop(0, NOUT)
        def _(t):
            row = rowbase + t * GRP
            pltpu.sync_copy(eidx_hbm.at[pl.ds(row, GRP)], gid2)
            pltpu.sync_copy(dst_hbm.at[pl.ds(row, GRP)], dst2)
            cps = [None] * GRP
            for c in range(SUB):
                cps[c] = pltpu.async_copy(
                    table_hbm.at[gid2.at[c]], rows[c % SUB], sems[c % SUB])
            for c in range(GRP):
                cps[c].wait()
                pltpu.sync_copy(rows[c % SUB], acc.at[dst2.at[c]], add=True)
                if c + SUB < GRP:
                    cps[c + SUB] = pltpu.async_copy(
                        table_hbm.at[gid2.at[c + SUB]], rows[c % SUB],
                        sems[c % SUB])

        plsc.subcore_barrier()
        row0 = sid * RPT
        prev = [None, None]
        for i, (off, sz) in enumerate(_RPT_CHUNKS):
            b = rows[i % 2]
            if prev[i % 2] is not None:
                prev[i % 2].wait()
            pltpu.sync_copy(acc.at[pl.ds(row0 + off, sz)], b.at[pl.ds(0, sz)])
            prev[i % 2] = pltpu.async_copy(
                b.at[pl.ds(0, sz)], out_hbm.at[cid, pl.ds(row0 + off, sz)],
                sems[i % 2])
        for p in prev:
            if p is not None:
                p.wait()

    return k


@functools.cache
def _sc_aug():
    @functools.partial(
        pl.kernel,
        out_type=jax.ShapeDtypeStruct((NC, NPAD, D), jnp.float32),
        mesh=_mesh(),
        scratch_types=[
            pltpu.VMEM((GRP, CH), jnp.int32),
            pltpu.VMEM((CH // 4, D), jnp.float32),
            pltpu.VMEM((CH // 4, D), jnp.float32),
            pltpu.VMEM((CH, D), jnp.float32),
            pltpu.VMEM((CH, D), jnp.float32),
            pltpu.VMEM_SHARED((NPAD, D), jnp.float32),
            pltpu.SemaphoreType.DMA,
            pltpu.SemaphoreType.DMA,
            pltpu.SemaphoreType.DMA,
            pltpu.SemaphoreType.DMA,
        ],
    )
    def k(aug4_hbm, dst_hbm, out_hbm, dst2, pk0, pk1, st0, st1, acc,
          sp0, sp1, ss0, ss1):
        pks = [pk0, pk1]
        sts = [st0, st1]
        psem = [sp0, sp1]
        ssem = [ss0, ss1]
        cid = lax.axis_index("c")
        sid = lax.axis_index("s")
        wid = sid * NC + cid
        _acc_zero_and_sync(acc, st0, D, sid)
        _zero_buf(st1, D)
        rowbase = wid * ROWS_PT

        @pl.loop(0, NOUT)
        def _(t):
            row = rowbase + t * GRP
            pltpu.sync_copy(dst_hbm.at[pl.ds(row, GRP)], dst2)
            cp = [None] * GRP
            cs = [None] * GRP
            for c in range(2):
                cp[c] = pltpu.async_copy(
                    aug4_hbm.at[pl.ds((row + c) * (CH // 4), CH // 4)],
                    pks[c % 2], psem[c % 2])
            for c in range(GRP):
                b = c % 2
                cp[c].wait()
                if c >= 2:
                    cs[c - 2].wait()
                pk = pks[b]
                st = sts[b]

                @pl.loop(0, CH // 4)
                def _(r):
                    for a in range(4):
                        st[a * 32 + r, pl.ds(0, 16)] = pk[r, pl.ds(a * 32, 16)]
                        st[a * 32 + r, pl.ds(16, 16)] = pk[r, pl.ds(a * 32 + 16, 16)]
                cs[c] = pltpu.async_copy(st, acc.at[dst2.at[c]], ssem[b],
                                         add=True)
                if c + 2 < GRP:
                    cp[c + 2] = pltpu.async_copy(
                        aug4_hbm.at[pl.ds((row + c + 2) * (CH // 4), CH // 4)],
                        pks[b], psem[b])
            cs[GRP - 2].wait()
            cs[GRP - 1].wait()

        plsc.subcore_barrier()
        row0 = sid * RPT
        for off, sz in _RPT_CHUNKS:
            pltpu.sync_copy(acc.at[pl.ds(row0 + off, sz)], st0.at[pl.ds(0, sz)])
            pltpu.sync_copy(st0.at[pl.ds(0, sz)],
                            out_hbm.at[cid, pl.ds(row0 + off, sz)])

    return k


@functools.cache
def _sc_head():
    @functools.partial(
        pl.kernel,
        out_type=(
            jax.ShapeDtypeStruct((BK, D), jnp.float32),
            jax.ShapeDtypeStruct((BK, D), jnp.float32),
        ),
        mesh=_mesh(),
        scratch_types=[
            pltpu.VMEM((BKT,), jnp.int32),
            pltpu.VMEM((BKT, D), jnp.float32),
            pltpu.SemaphoreType.DMA,
        ],
    )
    def k(x_hbm, si_hbm, ti_hbm, so_hbm, to_hbm, idx_v, rows_v, sem):
        cid = lax.axis_index("c")
        sid = lax.axis_index("s")
        wid = sid * NC + cid
        base = wid * BKT
        pltpu.sync_copy(si_hbm.at[pl.ds(base, BKT)], idx_v)
        pltpu.async_copy(x_hbm.at[idx_v], rows_v, sem).wait()
        pltpu.sync_copy(rows_v, so_hbm.at[pl.ds(base, BKT)])
        pltpu.sync_copy(ti_hbm.at[pl.ds(base, BKT)], idx_v)
        pltpu.async_copy(x_hbm.at[idx_v], rows_v, sem).wait()
        pltpu.sync_copy(rows_v, to_hbm.at[pl.ds(base, BKT)])

    return k


def _layernorm(out, lns, lnb):
    mu = jnp.mean(out, axis=1, keepdims=True)
    var = jnp.mean((out - mu) ** 2, axis=1, keepdims=True)
    return (out - mu) * lax.rsqrt(var + 1e-5) * lns + lnb


def _tc0_body(part_ref, naug_ref, wself0_ref, wedge0_ref, bias_ref, lns_ref,
              lnb_ref, wreln_ref, x_ref, trans_ref):
    p = naug_ref[0] + naug_ref[1]
    eagg = p[:, :EDIM]
    inv = 1.0 / jnp.maximum(p[:, EDIM:EDIM + 1], 1.0)
    s = part_ref[0] + part_ref[1]
    agg = s + jnp.dot(eagg, wedge0_ref[...], preferred_element_type=jnp.float32)
    out = agg * inv + jnp.sum(wself0_ref[...], axis=0, keepdims=True) + bias_ref[...]
    x = jnp.maximum(_layernorm(out, lns_ref[...], lnb_ref[...]), 0.0) + 1.0
    x_ref[...] = x
    for r in range(R):
        trans_ref[r] = jnp.dot(x, wreln_ref[r], preferred_element_type=jnp.float32)


def _tcl_body(has_next, part_ref, naug_ref, x_ref, wself_ref, wedge_ref,
              bias_ref, lns_ref, lnb_ref, *rest):
    if has_next:
        wreln_ref, xo_ref, trans_ref = rest
    else:
        wreln_ref, trans_ref = None, None
        (xo_ref,) = rest
    p = naug_ref[0] + naug_ref[1]
    eagg = p[:, :EDIM]
    inv = 1.0 / jnp.maximum(p[:, EDIM:EDIM + 1], 1.0)
    s = part_ref[0] + part_ref[1]
    agg = s + jnp.dot(eagg, wedge_ref[...], preferred_element_type=jnp.float32)
    x = x_ref[...]
    out = agg * inv + jnp.dot(x, wself_ref[...], preferred_element_type=jnp.float32) + bias_ref[...]
    xn = jnp.maximum(_layernorm(out, lns_ref[...], lnb_ref[...]), 0.0) + x
    xo_ref[...] = xn
    if has_next:
        for r in range(R):
            trans_ref[r] = jnp.dot(xn, wreln_ref[r], preferred_element_type=jnp.float32)


def _full3(shape):
    return pl.BlockSpec(shape, lambda i: tuple(0 for _ in shape))


def _tc0(part, naug, wself0, wedge0, bias0, lns0, lnb0, wrel1):
    return pl.pallas_call(
        _tc0_body,
        grid=(GRID_N,),
        in_specs=[
            pl.BlockSpec((NC, BN, D), lambda i: (0, i, 0)),
            pl.BlockSpec((NC, BN, AW), lambda i: (0, i, 0)),
            _full3((D, D)),
            _full3((EDIM, D)),
            _full3((1, D)),
            _full3((1, D)),
            _full3((1, D)),
            _full3((R, D, D)),
        ],
        out_specs=[
            pl.BlockSpec((BN, D), lambda i: (i, 0)),
            pl.BlockSpec((R, BN, D), lambda i: (0, i, 0)),
        ],
        out_shape=[
            jax.ShapeDtypeStruct((N_NODES, D), jnp.float32),
            jax.ShapeDtypeStruct((R, N_NODES, D), jnp.float32),
        ],
    )(part, naug, wself0, wedge0, bias0, lns0, lnb0, wrel1)


def _tc_layer(part, naug, x, wself, wedge, biasl, lns, lnb, wreln=None):
    has_next = wreln is not None
    in_specs = [
        pl.BlockSpec((NC, BN, D), lambda i: (0, i, 0)),
        pl.BlockSpec((NC, BN, AW), lambda i: (0, i, 0)),
        pl.BlockSpec((BN, D), lambda i: (i, 0)),
        _full3((D, D)),
        _full3((EDIM, D)),
        _full3((1, D)),
        _full3((1, D)),
        _full3((1, D)),
    ]
    out_specs = [pl.BlockSpec((BN, D), lambda i: (i, 0))]
    out_shape = [jax.ShapeDtypeStruct((N_NODES, D), jnp.float32)]
    args = [part, naug, x, wself, wedge, biasl, lns, lnb]
    if has_next:
        in_specs.append(_full3((R, D, D)))
        out_specs.append(pl.BlockSpec((R, BN, D), lambda i: (0, i, 0)))
        out_shape.append(jax.ShapeDtypeStruct((R, N_NODES, D), jnp.float32))
        args.append(wreln)
    return pl.pallas_call(
        functools.partial(_tcl_body, has_next),
        grid=(GRID_N,),
        in_specs=in_specs,
        out_specs=out_specs,
        out_shape=out_shape,
    )(*args)


QP = EP // 4
PBLK = 2560
_NEBLK = E_EDGES // PBLK - 1


def _pack_body(e0, e1, e2, e3, out_ref):
    i = pl.program_id(0)
    pieces = []
    for a, eref in enumerate((e0, e1, e2, e3)):
        base = i * PBLK + a * QP
        gr = base + lax.broadcasted_iota(jnp.int32, (PBLK, 1), 0)
        valid = (gr < E_EDGES).astype(jnp.float32)
        pieces.append(eref[...] * valid)
        pieces.append(valid)
        pieces.append(jnp.zeros((PBLK, 15), jnp.float32))
    out_ref[...] = jnp.concatenate(pieces, axis=1)


def _tc_pack(emb):
    def _emap(a):
        return lambda i: (jnp.minimum(i + a * (QP // PBLK), _NEBLK), 0)

    espec = [pl.BlockSpec((PBLK, EDIM), _emap(a)) for a in range(4)]
    return pl.pallas_call(
        _pack_body,
        grid=(QP // PBLK,),
        in_specs=espec,
        out_specs=pl.BlockSpec((PBLK, D), lambda i: (i, 0)),
        out_shape=jax.ShapeDtypeStruct((QP, D), jnp.float32),
    )(emb, emb, emb, emb)


MREP = 1024


def _m0_body(w_ref, out_ref):
    s = jnp.sum(w_ref[0], axis=0, keepdims=True)
    out_ref[...] = jnp.broadcast_to(s, (MREP, D))


def _tc_m0(wrel0):
    return pl.pallas_call(
        _m0_body,
        grid=(R,),
        in_specs=[pl.BlockSpec((1, D, D), lambda r: (r, 0, 0))],
        out_specs=pl.BlockSpec((MREP, D), lambda r: (r, 0)),
        out_shape=jax.ShapeDtypeStruct((R * MREP, D), jnp.float32),
    )(wrel0)


def _tc_head_body(se_ref, te_ref, oh_ref, rel_ref, out_ref):
    rel_e = jnp.dot(oh_ref[...], rel_ref[...], preferred_element_type=jnp.float32)
    out_ref[...] = jnp.sum(se_ref[...] * te_ref[...] * rel_e, axis=1, keepdims=True)


def _tc_head(se, te, oh, rel_emb):
    return pl.pallas_call(
        _tc_head_body,
        grid=(1,),
        in_specs=[
            _full3((BK, D)),
            _full3((BK, D)),
            _full3((BK, R)),
            _full3((R, D)),
        ],
        out_specs=pl.BlockSpec((BK, 1), lambda i: (0, 0)),
        out_shape=jax.ShapeDtypeStruct((BK, 1), jnp.float32),
    )(se, te, oh, rel_emb)


def kernel(edge_embeddings, Wrel, Wself, bias, Wedge, ln_scale, ln_beta,
           rel_emb, edge_index, edge_type, batch):
    f32 = jnp.float32
    src = edge_index[0]
    dst = edge_index[1]
    et = edge_type
    eidx = et * N_NODES + src

    npad_extra = EP - E_EDGES
    ar = lax.iota(jnp.int32, npad_extra)
    eidx_p = jnp.concatenate([eidx, (ar * 97) % (R * N_NODES)])
    dst_p = jnp.concatenate([dst, N_NODES + (ar % (NPAD - N_NODES))])

    eidx2 = eidx_p.reshape(EP // CH, CH)
    dst2 = dst_p.reshape(EP // CH, CH)
    aug4 = _tc_pack(edge_embeddings)
    dstp4 = dst_p.reshape(4, EP // CH, CH // 4).transpose(1, 0, 2).reshape(
        EP // CH, CH)
    naug = _sc_aug()(aug4, dstp4)

    m0t = _tc_m0(Wrel[0])
    idx0 = et * MREP + (lax.iota(jnp.int32, E_EDGES) % MREP)
    idx02 = jnp.concatenate(
        [idx0, (ar * 97) % (R * MREP)]).reshape(EP // CH, CH)
    part0 = _sc_seg()(m0t, idx02, dst2)
    x, trans = _tc0(part0, naug, Wself[0], Wedge[0], bias[0][None],
                    ln_scale[0][None], ln_beta[0][None], Wrel[1])
    for l in range(1, NLAYERS):
        part = _sc_seg()(trans.reshape(R * N_NODES, D), eidx2, dst2)
        if l < NLAYERS - 1:
            x, trans = _tc_layer(part, naug, x, Wself[l], Wedge[l], bias[l][None],
                                 ln_scale[l][None], ln_beta[l][None], Wrel[l + 1])
        else:
            (x,) = _tc_layer(part, naug, x, Wself[l], Wedge[l], bias[l][None],
                             ln_scale[l][None], ln_beta[l][None])

    bs = batch[:, :, 0].reshape(-1)
    bt = batch[:, :, 1].reshape(-1)
    br = batch[:, :, 2].reshape(-1)
    se, te = _sc_head()(x, bs, bt)
    oh = (br[:, None] == lax.iota(jnp.int32, R)[None, :]).astype(f32)
    score = _tc_head(se, te, oh, rel_emb)
    return score[:, 0].reshape(B, K)


# speedup vs baseline: 1.2899x; 1.0362x over previous
import functools

import jax
import jax.numpy as jnp
from jax import lax
from jax.experimental import pallas as pl
from jax.experimental.pallas import tpu as pltpu
from jax.experimental.pallas import tpu_sc as plsc

N_NODES = 10000
E_EDGES = 320000
D = 128
R = 8
NLAYERS = 6
EDIM = 16
B = 64
K = 32

NC = 2
NS = 16
NW = NC * NS
CH = 128
TPB = 10240
EP = NW * TPB
NPAD = N_NODES + 112
RPT = NPAD // NS
AW = 128
BK = B * K
BKT = BK // NW

BN = 1000
GRID_N = N_NODES // BN

@functools.cache
def _mesh():
    return plsc.VectorSubcoreMesh(
        core_axis_name="c", subcore_axis_name="s", num_cores=NC, num_subcores=NS)

_RPT_CHUNKS = [(0, 128), (128, 128), (256, 128), (384, 128), (512, 120)]


def _zero_buf(buf, width):
    z = jnp.zeros((16,), jnp.float32)

    @pl.loop(0, CH)
    def _(i):
        for j in range(width // 16):
            buf[i, pl.ds(j * 16, 16)] = z


def _acc_zero_and_sync(acc, buf, width, sid):
    _zero_buf(buf, width)
    row0 = sid * RPT
    for off, sz in _RPT_CHUNKS:
        pltpu.sync_copy(buf.at[pl.ds(0, sz)], acc.at[pl.ds(row0 + off, sz)])
    plsc.subcore_barrier()


GRP = 16
SUB = 2
ROWS_PT = TPB // CH
NOUT = ROWS_PT // GRP


@functools.cache
def _sc_seg():
    @functools.partial(
        pl.kernel,
        out_type=jax.ShapeDtypeStruct((NC, NPAD, D), jnp.float32),
        mesh=_mesh(),
        scratch_types=[
            pltpu.VMEM((GRP, CH), jnp.int32),
            pltpu.VMEM((GRP, CH), jnp.int32),
            pltpu.VMEM((CH, D), jnp.float32),
            pltpu.VMEM((CH, D), jnp.float32),
            pltpu.VMEM_SHARED((NPAD, D), jnp.float32),
            pltpu.SemaphoreType.DMA,
            pltpu.SemaphoreType.DMA,
        ],
    )
    def k(table_hbm, eidx_hbm, dst_hbm, out_hbm, gid2, dst2, r0, r1,
          acc, s0, s1):
        rows = [r0, r1]
        sems = [s0, s1]
        cid = lax.axis_index("c")
        sid = lax.axis_index("s")
        wid = sid * NC + cid
        _acc_zero_and_sync(acc, r0, D, sid)
        rowbase = wid * ROWS_PT

        @pl.loop(0, NOUT)
        def _(t):
            row = rowbase + t * GRP
            pltpu.sync_copy(eidx_hbm.at[pl.ds(row, GRP)], gid2)
            pltpu.sync_copy(dst_hbm.at[pl.ds(row, GRP)], dst2)
            cps = [None] * GRP
            for c in range(SUB):
                cps[c] = pltpu.async_copy(
                    table_hbm.at[gid2.at[c]], rows[c % SUB], sems[c % SUB])
            for c in range(GRP):
                cps[c].wait()
                pltpu.sync_copy(rows[c % SUB], acc.at[dst2.at[c]], add=True)
                if c + SUB < GRP:
                    cps[c + SUB] = pltpu.async_copy(
                        table_hbm.at[gid2.at[c + SUB]], rows[c % SUB],
                        sems[c % SUB])

        plsc.subcore_barrier()
        row0 = sid * RPT
        prev = [None, None]
        for i, (off, sz) in enumerate(_RPT_CHUNKS):
            b = rows[i % 2]
            if prev[i % 2] is not None:
                prev[i % 2].wait()
            pltpu.sync_copy(acc.at[pl.ds(row0 + off, sz)], b.at[pl.ds(0, sz)])
            prev[i % 2] = pltpu.async_copy(
                b.at[pl.ds(0, sz)], out_hbm.at[cid, pl.ds(row0 + off, sz)],
                sems[i % 2])
        for p in prev:
            if p is not None:
                p.wait()

    return k


@functools.cache
def _sc_aug():
    @functools.partial(
        pl.kernel,
        out_type=jax.ShapeDtypeStruct((NC, NPAD, D), jnp.float32),
        mesh=_mesh(),
        scratch_types=[
            pltpu.VMEM((GRP, CH), jnp.int32),
            pltpu.VMEM((CH // 4, D), jnp.float32),
            pltpu.VMEM((CH // 4, D), jnp.float32),
            pltpu.VMEM((CH, D), jnp.float32),
            pltpu.VMEM((CH, D), jnp.float32),
            pltpu.VMEM_SHARED((NPAD, D), jnp.float32),
            pltpu.SemaphoreType.DMA,
            pltpu.SemaphoreType.DMA,
            pltpu.SemaphoreType.DMA,
            pltpu.SemaphoreType.DMA,
        ],
    )
    def k(aug4_hbm, dst_hbm, out_hbm, dst2, pk0, pk1, st0, st1, acc,
          sp0, sp1, ss0, ss1):
        pks = [pk0, pk1]
        sts = [st0, st1]
        psem = [sp0, sp1]
        ssem = [ss0, ss1]
        cid = lax.axis_index("c")
        sid = lax.axis_index("s")
        wid = sid * NC + cid
        _acc_zero_and_sync(acc, st0, D, sid)
        _zero_buf(st1, D)
        rowbase = wid * ROWS_PT

        @pl.loop(0, NOUT)
        def _(t):
            row = rowbase + t * GRP
            pltpu.sync_copy(dst_hbm.at[pl.ds(row, GRP)], dst2)
            cp = [None] * GRP
            cs = [None] * GRP
            for c in range(2):
                cp[c] = pltpu.async_copy(
                    aug4_hbm.at[pl.ds((row + c) * (CH // 4), CH // 4)],
                    pks[c % 2], psem[c % 2])
            for c in range(GRP):
                b = c % 2
                cp[c].wait()
                if c >= 2:
                    cs[c - 2].wait()
                pk = pks[b]
                st = sts[b]

                @pl.loop(0, CH // 4)
                def _(r):
                    for a in range(4):
                        st[a * 32 + r, pl.ds(0, 16)] = pk[r, pl.ds(a * 32, 16)]
                        st[a * 32 + r, pl.ds(16, 16)] = pk[r, pl.ds(a * 32 + 16, 16)]
                cs[c] = pltpu.async_copy(st, acc.at[dst2.at[c]], ssem[b],
                                         add=True)
                if c + 2 < GRP:
                    cp[c + 2] = pltpu.async_copy(
                        aug4_hbm.at[pl.ds((row + c + 2) * (CH // 4), CH // 4)],
                        pks[b], psem[b])
            cs[GRP - 2].wait()
            cs[GRP - 1].wait()

        plsc.subcore_barrier()
        row0 = sid * RPT
        for off, sz in _RPT_CHUNKS:
            pltpu.sync_copy(acc.at[pl.ds(row0 + off, sz)], st0.at[pl.ds(0, sz)])
            pltpu.sync_copy(st0.at[pl.ds(0, sz)],
                            out_hbm.at[cid, pl.ds(row0 + off, sz)])

    return k


@functools.cache
def _sc_head():
    @functools.partial(
        pl.kernel,
        out_type=(
            jax.ShapeDtypeStruct((BK, D), jnp.float32),
            jax.ShapeDtypeStruct((BK, D), jnp.float32),
        ),
        mesh=_mesh(),
        scratch_types=[
            pltpu.VMEM((BKT,), jnp.int32),
            pltpu.VMEM((BKT, D), jnp.float32),
            pltpu.SemaphoreType.DMA,
        ],
    )
    def k(x_hbm, si_hbm, ti_hbm, so_hbm, to_hbm, idx_v, rows_v, sem):
        cid = lax.axis_index("c")
        sid = lax.axis_index("s")
        wid = sid * NC + cid
        base = wid * BKT
        pltpu.sync_copy(si_hbm.at[pl.ds(base, BKT)], idx_v)
        pltpu.async_copy(x_hbm.at[idx_v], rows_v, sem).wait()
        pltpu.sync_copy(rows_v, so_hbm.at[pl.ds(base, BKT)])
        pltpu.sync_copy(ti_hbm.at[pl.ds(base, BKT)], idx_v)
        pltpu.async_copy(x_hbm.at[idx_v], rows_v, sem).wait()
        pltpu.sync_copy(rows_v, to_hbm.at[pl.ds(base, BKT)])

    return k


def _layernorm(out, lns, lnb):
    mu = jnp.mean(out, axis=1, keepdims=True)
    var = jnp.mean((out - mu) ** 2, axis=1, keepdims=True)
    return (out - mu) * lax.rsqrt(var + 1e-5) * lns + lnb


def _tc0_body(part_ref, naug_ref, wself0_ref, wedge0_ref, bias_ref, lns_ref,
              lnb_ref, wreln_ref, x_ref, trans_ref):
    p = naug_ref[0] + naug_ref[1]
    eagg = p[:, :EDIM]
    inv = 1.0 / jnp.maximum(p[:, EDIM:EDIM + 1], 1.0)
    s = part_ref[0] + part_ref[1]
    agg = s + jnp.dot(eagg, wedge0_ref[...], preferred_element_type=jnp.float32)
    out = agg * inv + jnp.sum(wself0_ref[...], axis=0, keepdims=True) + bias_ref[...]
    x = jnp.maximum(_layernorm(out, lns_ref[...], lnb_ref[...]), 0.0) + 1.0
    x_ref[...] = x
    for r in range(R):
        trans_ref[r] = jnp.dot(x, wreln_ref[r], preferred_element_type=jnp.float32)


def _tcl_body(has_next, part_ref, naug_ref, x_ref, wself_ref, wedge_ref,
              bias_ref, lns_ref, lnb_ref, *rest):
    if has_next:
        wreln_ref, xo_ref, trans_ref = rest
    else:
        wreln_ref, trans_ref = None, None
        (xo_ref,) = rest
    p = naug_ref[0] + naug_ref[1]
    eagg = p[:, :EDIM]
    inv = 1.0 / jnp.maximum(p[:, EDIM:EDIM + 1], 1.0)
    s = part_ref[0] + part_ref[1]
    agg = s + jnp.dot(eagg, wedge_ref[...], preferred_element_type=jnp.float32)
    x = x_ref[...]
    out = agg * inv + jnp.dot(x, wself_ref[...], preferred_element_type=jnp.float32) + bias_ref[...]
    xn = jnp.maximum(_layernorm(out, lns_ref[...], lnb_ref[...]), 0.0) + x
    xo_ref[...] = xn
    if has_next:
        for r in range(R):
            trans_ref[r] = jnp.dot(xn, wreln_ref[r], preferred_element_type=jnp.float32)


def _full3(shape):
    return pl.BlockSpec(shape, lambda i: tuple(0 for _ in shape))


def _tc0(part, naug, wself0, wedge0, bias0, lns0, lnb0, wrel1):
    return pl.pallas_call(
        _tc0_body,
        grid=(GRID_N,),
        in_specs=[
            pl.BlockSpec((NC, BN, D), lambda i: (0, i, 0)),
            pl.BlockSpec((NC, BN, AW), lambda i: (0, i, 0)),
            _full3((D, D)),
            _full3((EDIM, D)),
            _full3((1, D)),
            _full3((1, D)),
            _full3((1, D)),
            _full3((R, D, D)),
        ],
        out_specs=[
            pl.BlockSpec((BN, D), lambda i: (i, 0)),
            pl.BlockSpec((R, BN, D), lambda i: (0, i, 0)),
        ],
        out_shape=[
            jax.ShapeDtypeStruct((N_NODES, D), jnp.float32),
            jax.ShapeDtypeStruct((R, N_NODES, D), jnp.float32),
        ],
    )(part, naug, wself0, wedge0, bias0, lns0, lnb0, wrel1)


def _tc_layer(part, naug, x, wself, wedge, biasl, lns, lnb, wreln=None):
    has_next = wreln is not None
    in_specs = [
        pl.BlockSpec((NC, BN, D), lambda i: (0, i, 0)),
        pl.BlockSpec((NC, BN, AW), lambda i: (0, i, 0)),
        pl.BlockSpec((BN, D), lambda i: (i, 0)),
        _full3((D, D)),
        _full3((EDIM, D)),
        _full3((1, D)),
        _full3((1, D)),
        _full3((1, D)),
    ]
    out_specs = [pl.BlockSpec((BN, D), lambda i: (i, 0))]
    out_shape = [jax.ShapeDtypeStruct((N_NODES, D), jnp.float32)]
    args = [part, naug, x, wself, wedge, biasl, lns, lnb]
    if has_next:
        in_specs.append(_full3((R, D, D)))
        out_specs.append(pl.BlockSpec((R, BN, D), lambda i: (0, i, 0)))
        out_shape.append(jax.ShapeDtypeStruct((R, N_NODES, D), jnp.float32))
        args.append(wreln)
    return pl.pallas_call(
        functools.partial(_tcl_body, has_next),
        grid=(GRID_N,),
        in_specs=in_specs,
        out_specs=out_specs,
        out_shape=out_shape,
    )(*args)


QP = EP // 4
PBLK = 2560
_NEBLK = E_EDGES // PBLK - 1


def _pack_body(e0, e1, e2, e3, out_ref):
    i = pl.program_id(0)
    pieces = []
    for a, eref in enumerate((e0, e1, e2, e3)):
        base = i * PBLK + a * QP
        gr = base + lax.broadcasted_iota(jnp.int32, (PBLK, 1), 0)
        valid = (gr < E_EDGES).astype(jnp.float32)
        pieces.append(eref[...] * valid)
        pieces.append(valid)
        pieces.append(jnp.zeros((PBLK, 15), jnp.float32))
    out_ref[...] = jnp.concatenate(pieces, axis=1)


def _tc_pack(emb):
    def _emap(a):
        return lambda i: (jnp.minimum(i + a * (QP // PBLK), _NEBLK), 0)

    espec = [pl.BlockSpec((PBLK, EDIM), _emap(a)) for a in range(4)]
    return pl.pallas_call(
        _pack_body,
        grid=(QP // PBLK,),
        in_specs=espec,
        out_specs=pl.BlockSpec((PBLK, D), lambda i: (i, 0)),
        out_shape=jax.ShapeDtypeStruct((QP, D), jnp.float32),
    )(emb, emb, emb, emb)


MREP = 1024


def _m0_body(w_ref, out_ref):
    s = jnp.sum(w_ref[0], axis=0, keepdims=True)
    out_ref[...] = jnp.broadcast_to(s, (MREP, D))


def _tc_m0(wrel0):
    return pl.pallas_call(
        _m0_body,
        grid=(R,),
        in_specs=[pl.BlockSpec((1, D, D), lambda r: (r, 0, 0))],
        out_specs=pl.BlockSpec((MREP, D), lambda r: (r, 0)),
        out_shape=jax.ShapeDtypeStruct((R * MREP, D), jnp.float32),
    )(wrel0)


def _tc_head_body(se_ref, te_ref, oh_ref, rel_ref, out_ref):
    rel_e = jnp.dot(oh_ref[...], rel_ref[...], preferred_element_type=jnp.float32)
    out_ref[...] = jnp.sum(se_ref[...] * te_ref[...] * rel_e, axis=1, keepdims=True)


def _tc_head(se, te, oh, rel_emb):
    return pl.pallas_call(
        _tc_head_body,
        grid=(1,),
        in_specs=[
            _full3((BK, D)),
            _full3((BK, D)),
            _full3((BK, R)),
            _full3((R, D)),
        ],
        out_specs=pl.BlockSpec((BK, 1), lambda i: (0, 0)),
        out_shape=jax.ShapeDtypeStruct((BK, 1), jnp.float32),
    )(se, te, oh, rel_emb)


def kernel(edge_embeddings, Wrel, Wself, bias, Wedge, ln_scale, ln_beta,
           rel_emb, edge_index, edge_type, batch):
    f32 = jnp.float32
    src = edge_index[0]
    dst = edge_index[1]
    et = edge_type
    eidx = et * N_NODES + src

    npad_extra = EP - E_EDGES
    ar = lax.iota(jnp.int32, npad_extra)
    eidx_p = jnp.concatenate([eidx, (ar * 97) % (R * N_NODES)])
    dst_p = jnp.concatenate([dst, N_NODES + (ar % (NPAD - N_NODES))])

    eidx2 = eidx_p.reshape(EP // CH, CH)
    dst2 = dst_p.reshape(EP // CH, CH)
    aug4 = _tc_pack(edge_embeddings)
    dstp4 = dst_p.reshape(4, EP // CH, CH // 4).transpose(1, 0, 2).reshape(
        EP // CH, CH)
    naug = _sc_aug()(aug4, dstp4)

    m0t = _tc_m0(Wrel[0])
    idx0 = et * MREP + (lax.iota(jnp.int32, E_EDGES) % MREP)
    idx02 = jnp.concatenate(
        [idx0, (ar * 97) % (R * MREP)]).reshape(EP // CH, CH)
    part0 = _sc_seg()(m0t, idx02, dst2)
    x, trans = _tc0(part0, naug, Wself[0], Wedge[0], bias[0][None],
                    ln_scale[0][None], ln_beta[0][None], Wrel[1])
    for l in range(1, NLAYERS):
        part = _sc_seg()(trans.reshape(R * N_NODES, D), eidx2, dst2)
        if l < NLAYERS - 1:
            x, trans = _tc_layer(part, naug, x, Wself[l], Wedge[l], bias[l][None],
                                 ln_scale[l][None], ln_beta[l][None], Wrel[l + 1])
        else:
            (x,) = _tc_layer(part, naug, x, Wself[l], Wedge[l], bias[l][None],
                             ln_scale[l][None], ln_beta[l][None])

    bs = batch[:, :, 0].reshape(-1)
    bt = batch[:, :, 1].reshape(-1)
    br = batch[:, :, 2].reshape(-1)
    se, te = _sc_head()(x, bs, bt)
    oh = (br[:, None] == lax.iota(jnp.int32, R)[None, :]).astype(f32)
    score = _tc_head(se, te, oh, rel_emb)
    return score[:, 0].reshape(B, K)
